# baseline (speedup 1.0000x reference)
import functools

import jax
import jax.numpy as jnp
from jax import lax
from jax.experimental import pallas as pl
from jax.experimental.pallas import tpu as pltpu
from jax.experimental.pallas import tpu_sc as plsc

_B = 16384
_SEQ = 200
_D = 16
_H = 128
_NW = 32
_HALF = 100
_CB = 8


_EC = 2
_RC = 2 * _EC
_NCH = (_B // _NW) // _EC
_NBUF = 4


def _sc_pool(table, x2):
    elems_per_w = _B // _NW
    mesh = plsc.VectorSubcoreMesh(core_axis_name="c", subcore_axis_name="s")

    @functools.partial(
        pl.kernel,
        out_type=jax.ShapeDtypeStruct((_B, _D), jnp.float32),
        mesh=mesh,
        scratch_types=(
            [pltpu.VMEM((_RC, _HALF), jnp.int32) for _ in range(_NBUF)]
            + [pltpu.VMEM((_RC * _HALF, 2 * _D), jnp.float32) for _ in range(_NBUF)]
            + [pltpu.VMEM((elems_per_w, _D), jnp.float32)]
            + [pltpu.SemaphoreType.DMA for _ in range(_NBUF)]
        ),
        compiler_params=pltpu.CompilerParams(use_tc_tiling_on_sc=False),
    )
    def k(table_hbm, x_hbm, out_hbm, *refs):
        idxb = refs[:_NBUF]
        rows = refs[_NBUF:2 * _NBUF]
        pooled_all = refs[2 * _NBUF]
        sems = refs[2 * _NBUF + 1:]
        wid = lax.axis_index("s") * 2 + lax.axis_index("c")
        row0 = wid * (elems_per_w * 2)
        elem0 = wid * elems_per_w

        def fire(ci, b):
            pltpu.sync_copy(x_hbm.at[pl.ds(row0 + ci * _RC, _RC)], idxb[b])
            for j in range(_RC):
                pltpu.async_copy(
                    table_hbm.at[idxb[b].at[j]],
                    rows[b].at[pl.ds(j * _HALF, _HALF)],
                    sems[b],
                )

        def process(ci, b):
            r = rows[b]
            for j in range(_RC):
                pltpu.make_async_copy(
                    table_hbm.at[pl.ds(0, _HALF)],
                    r.at[pl.ds(j * _HALF, _HALF)],
                    sems[b],
                ).wait()

            def acc_body(t, accs):
                return tuple(
                    accs[i]
                    + r[i * _SEQ + 2 * t, :] + r[i * _SEQ + 2 * t + 1, :]
                    + r[i * _SEQ + _HALF + 2 * t, :]
                    + r[i * _SEQ + _HALF + 2 * t + 1, :]
                    for i in range(_EC)
                )

            accs = tuple(r[i * _SEQ, pl.ds(0, _D)] for i in range(_EC))
            for i in range(_EC):
                pooled_all[ci * _EC + i, :] = accs[i] * (1.0 / _SEQ)

        for b in range(_NBUF):
            fire(b, b)

        @pl.loop(0, _NCH - _NBUF, step=_NBUF)
        def _(cv):
            for b in range(_NBUF):
                process(cv + b, b)
                fire(cv + b + _NBUF, b)

        for b in range(_NBUF):
            process(_NCH - _NBUF + b, b)
        pltpu.sync_copy(pooled_all, out_hbm.at[pl.ds(elem0, elems_per_w)])

    return k(table, x2)


def _mlp(pooled, W1, b1, W2, b2):
    bm = 2048

    def body(p_ref, w1_ref, b1_ref, w2_ref, b2_ref, o_ref):
        h = jnp.dot(p_ref[...], w1_ref[...], preferred_element_type=jnp.float32)
        h = jnp.maximum(h + b1_ref[...], 0.0)
        z = jnp.dot(h, w2_ref[...], preferred_element_type=jnp.float32)
        o_ref[...] = jax.nn.sigmoid(z + b2_ref[...])

    return pl.pallas_call(
        body,
        grid=(_B // bm,),
        in_specs=[
            pl.BlockSpec((bm, _D), lambda i: (i, 0)),
            pl.BlockSpec((_D, _H), lambda i: (0, 0)),
            pl.BlockSpec((1, _H), lambda i: (0, 0)),
            pl.BlockSpec((_H, 1), lambda i: (0, 0)),
            pl.BlockSpec((1, 1), lambda i: (0, 0)),
        ],
        out_specs=pl.BlockSpec((bm, 1), lambda i: (i, 0)),
        out_shape=jax.ShapeDtypeStruct((_B, 1), jnp.float32),
    )(pooled, W1, b1, W2, b2)


@jax.jit
def kernel(x, emb_table, W1, b1, W2, b2):
    x2 = (x // 2).reshape(_B * 2, _HALF)
    t2 = emb_table[:1000000].reshape(500000, 2 * _D)
    pooled = _sc_pool(t2, x2)
    return _mlp(pooled, W1, b1.reshape(1, _H), W2, b2.reshape(1, 1))


# speedup vs baseline: 1.0942x; 1.0942x over previous
import functools

import jax
import jax.numpy as jnp
from jax import lax
from jax.experimental import pallas as pl
from jax.experimental.pallas import tpu as pltpu
from jax.experimental.pallas import tpu_sc as plsc

_B = 16384
_SEQ = 200
_D = 16
_H = 128
_NW = 32
_HALF = 100
_CB = 8


_EC = 4
_RC = 2 * _EC
_NCH = (_B // _NW) // _EC
_NBUF = 4
_QCH = _NCH // 4


def _sc_pool(table, x2):
    elems_per_w = _B // _NW
    mesh = plsc.VectorSubcoreMesh(core_axis_name="c", subcore_axis_name="s")

    @functools.partial(
        pl.kernel,
        out_type=jax.ShapeDtypeStruct((_B, _D), jnp.float32),
        mesh=mesh,
        scratch_types=(
            [pltpu.VMEM((_QCH * _RC, _HALF), jnp.int32) for _ in range(2)]
            + [pltpu.VMEM((_RC * _HALF, _D), jnp.float32) for _ in range(_NBUF)]
            + [pltpu.VMEM((elems_per_w, _D), jnp.float32)]
            + [pltpu.SemaphoreType.DMA for _ in range(_NBUF + 1)]
        ),
        compiler_params=pltpu.CompilerParams(use_tc_tiling_on_sc=False),
    )
    def k(table_hbm, x_hbm, out_hbm, *refs):
        idxb = refs[0:2]
        rows = refs[2:2 + _NBUF]
        pooled_all = refs[2 + _NBUF]
        sems = refs[3 + _NBUF:3 + 2 * _NBUF]
        sem_i = refs[3 + 2 * _NBUF]
        wid = lax.axis_index("s") * 2 + lax.axis_index("c")
        row0 = wid * (elems_per_w * 2)
        elem0 = wid * elems_per_w
        qrows = _QCH * _RC

        def start_idx(q, ib):
            pltpu.async_copy(
                x_hbm.at[pl.ds(row0 + q * qrows, qrows)], idxb[ib], sem_i)

        def wait_idx(ib):
            pltpu.make_async_copy(
                x_hbm.at[pl.ds(0, qrows)], idxb[ib], sem_i).wait()

        def fire(lc, b, ib):
            for j in range(_RC):
                pltpu.async_copy(
                    table_hbm.at[idxb[ib].at[lc * _RC + j]],
                    rows[b].at[pl.ds(j * _HALF, _HALF)],
                    sems[b],
                )

        def process(ci, b):
            r = rows[b]
            for j in range(_RC):
                pltpu.make_async_copy(
                    table_hbm.at[pl.ds(0, _HALF)],
                    r.at[pl.ds(j * _HALF, _HALF)],
                    sems[b],
                ).wait()

            def acc_body(t, accs):
                return tuple(
                    accs[i] + r[i * _SEQ + t, :] + r[i * _SEQ + _HALF + t, :]
                    for i in range(_EC)
                )

            accs = lax.fori_loop(
                0, _HALF, acc_body,
                tuple(jnp.zeros((_D,), jnp.float32) for _ in range(_EC)),
            )
            for i in range(_EC):
                pooled_all[ci * _EC + i, :] = accs[i] * (1.0 / _SEQ)

        start_idx(0, 0)
        wait_idx(0)
        start_idx(1, 1)
        for b in range(_NBUF):
            fire(b, b, 0)

        for q in range(4):
            base = q * _QCH
            ib = q % 2

            @pl.loop(0, _QCH - 2 * _NBUF, step=_NBUF)
            def _(cv, base=base, ib=ib):
                for b in range(_NBUF):
                    process(base + cv + b, b)
                    fire(cv + b + _NBUF, b, ib)

            for b in range(_NBUF):
                process(base + _QCH - 2 * _NBUF + b, b)
                fire(_QCH - _NBUF + b, b, ib)

            if q < 3:
                wait_idx(1 - ib)
                for b in range(_NBUF):
                    process(base + _QCH - _NBUF + b, b)
                    fire(b, b, 1 - ib)
                if q < 2:
                    start_idx(q + 2, ib)
            else:
                for b in range(_NBUF):
                    process(base + _QCH - _NBUF + b, b)

        pltpu.sync_copy(pooled_all, out_hbm.at[pl.ds(elem0, elems_per_w)])

    return k(table, x2)


def _mlp(pooled, W1, b1, W2, b2):
    bm = 2048

    def body(p_ref, w1_ref, b1_ref, w2_ref, b2_ref, o_ref):
        h = jnp.dot(p_ref[...], w1_ref[...], preferred_element_type=jnp.float32)
        h = jnp.maximum(h + b1_ref[...], 0.0)
        z = jnp.dot(h, w2_ref[...], preferred_element_type=jnp.float32)
        o_ref[...] = jax.nn.sigmoid(z + b2_ref[...])

    return pl.pallas_call(
        body,
        grid=(_B // bm,),
        in_specs=[
            pl.BlockSpec((bm, _D), lambda i: (i, 0)),
            pl.BlockSpec((_D, _H), lambda i: (0, 0)),
            pl.BlockSpec((1, _H), lambda i: (0, 0)),
            pl.BlockSpec((_H, 1), lambda i: (0, 0)),
            pl.BlockSpec((1, 1), lambda i: (0, 0)),
        ],
        out_specs=pl.BlockSpec((bm, 1), lambda i: (i, 0)),
        out_shape=jax.ShapeDtypeStruct((_B, 1), jnp.float32),
    )(pooled, W1, b1, W2, b2)


@jax.jit
def kernel(x, emb_table, W1, b1, W2, b2):
    x2 = x.reshape(_B * 2, _HALF)
    pooled = _sc_pool(emb_table, x2)
    return _mlp(pooled, W1, b1.reshape(1, _H), W2, b2.reshape(1, 1))


# speedup vs baseline: 1.1096x; 1.0141x over previous
import functools

import jax
import jax.numpy as jnp
from jax import lax
from jax.experimental import pallas as pl
from jax.experimental.pallas import tpu as pltpu
from jax.experimental.pallas import tpu_sc as plsc

_B = 16384
_SEQ = 200
_D = 16
_H = 128
_NW = 32
_HALF = 100
_CB = 8


_EC = 4
_RC = 2 * _EC
_NCH = (_B // _NW) // _EC
_NBUF = 4
_QCH = _NCH // 4


def _sc_pool(table, x2):
    elems_per_w = _B // _NW
    mesh = plsc.VectorSubcoreMesh(core_axis_name="c", subcore_axis_name="s")

    @functools.partial(
        pl.kernel,
        out_type=jax.ShapeDtypeStruct((_B, _D), jnp.float32),
        mesh=mesh,
        scratch_types=(
            [pltpu.VMEM((_QCH * _EC, _SEQ), jnp.int32) for _ in range(2)]
            + [pltpu.VMEM((_RC * _HALF, _D), jnp.float32) for _ in range(_NBUF)]
            + [pltpu.VMEM((elems_per_w, _D), jnp.float32)]
            + [pltpu.SemaphoreType.DMA for _ in range(_NBUF + 1)]
        ),
        compiler_params=pltpu.CompilerParams(use_tc_tiling_on_sc=False),
    )
    def k(table_hbm, x_hbm, out_hbm, *refs):
        idxb = refs[0:2]
        rows = refs[2:2 + _NBUF]
        pooled_all = refs[2 + _NBUF]
        sems = refs[3 + _NBUF:3 + 2 * _NBUF]
        sem_i = refs[3 + 2 * _NBUF]
        wid = lax.axis_index("s") * 2 + lax.axis_index("c")
        row0 = wid * elems_per_w
        elem0 = wid * elems_per_w
        qrows = _QCH * _EC

        def start_idx(q, ib):
            pltpu.async_copy(
                x_hbm.at[pl.ds(row0 + q * qrows, qrows)], idxb[ib], sem_i)

        def wait_idx(ib):
            pltpu.make_async_copy(
                x_hbm.at[pl.ds(0, qrows)], idxb[ib], sem_i).wait()

        def fire(lc, b, ib):
            for j in range(_EC):
                pltpu.async_copy(
                    table_hbm.at[idxb[ib].at[lc * _EC + j]],
                    rows[b].at[pl.ds(j * _SEQ, _SEQ)],
                    sems[b],
                )

        def process(ci, b):
            r = rows[b]
            for j in range(_EC):
                pltpu.make_async_copy(
                    table_hbm.at[pl.ds(0, _SEQ)],
                    r.at[pl.ds(j * _SEQ, _SEQ)],
                    sems[b],
                ).wait()

            def acc_body(t, accs):
                return tuple(
                    accs[i] + r[i * _SEQ + t, :] + r[i * _SEQ + _HALF + t, :]
                    for i in range(_EC)
                )

            accs = lax.fori_loop(
                0, _HALF, acc_body,
                tuple(jnp.zeros((_D,), jnp.float32) for _ in range(_EC)),
            )
            for i in range(_EC):
                pooled_all[ci * _EC + i, :] = accs[i] * (1.0 / _SEQ)

        start_idx(0, 0)
        wait_idx(0)
        start_idx(1, 1)
        for b in range(_NBUF):
            fire(b, b, 0)

        for q in range(4):
            base = q * _QCH
            ib = q % 2

            @pl.loop(0, _QCH - 2 * _NBUF, step=_NBUF)
            def _(cv, base=base, ib=ib):
                for b in range(_NBUF):
                    process(base + cv + b, b)
                    fire(cv + b + _NBUF, b, ib)

            for b in range(_NBUF):
                process(base + _QCH - 2 * _NBUF + b, b)
                fire(_QCH - _NBUF + b, b, ib)

            if q < 3:
                wait_idx(1 - ib)
                for b in range(_NBUF):
                    process(base + _QCH - _NBUF + b, b)
                    fire(b, b, 1 - ib)
                if q < 2:
                    start_idx(q + 2, ib)
            else:
                for b in range(_NBUF):
                    process(base + _QCH - _NBUF + b, b)

        pltpu.sync_copy(pooled_all, out_hbm.at[pl.ds(elem0, elems_per_w)])

    return k(table, x2)


def _mlp(pooled, W1, b1, W2, b2):
    bm = 2048

    def body(p_ref, w1_ref, b1_ref, w2_ref, b2_ref, o_ref):
        h = jnp.dot(p_ref[...], w1_ref[...], preferred_element_type=jnp.float32)
        h = jnp.maximum(h + b1_ref[...], 0.0)
        z = jnp.dot(h, w2_ref[...], preferred_element_type=jnp.float32)
        o_ref[...] = jax.nn.sigmoid(z + b2_ref[...])

    return pl.pallas_call(
        body,
        grid=(_B // bm,),
        in_specs=[
            pl.BlockSpec((bm, _D), lambda i: (i, 0)),
            pl.BlockSpec((_D, _H), lambda i: (0, 0)),
            pl.BlockSpec((1, _H), lambda i: (0, 0)),
            pl.BlockSpec((_H, 1), lambda i: (0, 0)),
            pl.BlockSpec((1, 1), lambda i: (0, 0)),
        ],
        out_specs=pl.BlockSpec((bm, 1), lambda i: (i, 0)),
        out_shape=jax.ShapeDtypeStruct((_B, 1), jnp.float32),
    )(pooled, W1, b1, W2, b2)


@jax.jit
def kernel(x, emb_table, W1, b1, W2, b2):
    pooled = _sc_pool(emb_table, x)
    return _mlp(pooled, W1, b1.reshape(1, _H), W2, b2.reshape(1, 1))


# speedup vs baseline: 1.1106x; 1.0010x over previous
import functools

import jax
import jax.numpy as jnp
from jax import lax
from jax.experimental import pallas as pl
from jax.experimental.pallas import tpu as pltpu
from jax.experimental.pallas import tpu_sc as plsc

_B = 16384
_SEQ = 200
_D = 16
_H = 128
_NW = 32
_HALF = 100
_EC = 4
_NCH = (_B // _NW) // _EC
_NBUF = 4
_QCH = _NCH // 4


def _sc_pool(table, x2):
    elems_per_w = _B // _NW
    mesh = plsc.VectorSubcoreMesh(core_axis_name="c", subcore_axis_name="s")

    @functools.partial(
        pl.kernel,
        out_type=jax.ShapeDtypeStruct((_B, _D), jnp.float32),
        mesh=mesh,
        scratch_types=(
            [pltpu.VMEM((_QCH * _EC, _SEQ), jnp.int32) for _ in range(2)]
            + [pltpu.VMEM((_EC * _SEQ, _D), jnp.float32) for _ in range(_NBUF)]
            + [pltpu.VMEM((elems_per_w, _D), jnp.float32)]
            + [pltpu.SemaphoreType.DMA for _ in range(_NBUF + 1)]
        ),
        compiler_params=pltpu.CompilerParams(use_tc_tiling_on_sc=False),
    )
    def k(table_hbm, x_hbm, out_hbm, *refs):
        idxb = refs[0:2]
        rows = refs[2:2 + _NBUF]
        pooled_all = refs[2 + _NBUF]
        sems = refs[3 + _NBUF:3 + 2 * _NBUF]
        sem_i = refs[3 + 2 * _NBUF]
        wid = lax.axis_index("s") * 2 + lax.axis_index("c")
        row0 = wid * elems_per_w
        elem0 = wid * elems_per_w
        qrows = _QCH * _EC

        def start_idx(q, ib):
            pltpu.async_copy(
                x_hbm.at[pl.ds(row0 + q * qrows, qrows)], idxb[ib], sem_i)

        def wait_idx(ib):
            pltpu.make_async_copy(
                x_hbm.at[pl.ds(0, qrows)], idxb[ib], sem_i).wait()

        def fire(lc, b, ib):
            for j in range(_EC):
                pltpu.async_copy(
                    table_hbm.at[idxb[ib].at[lc * _EC + j]],
                    rows[b].at[pl.ds(j * _SEQ, _SEQ)],
                    sems[b],
                )

        def process(ci, b):
            r = rows[b]
            for j in range(_EC):
                pltpu.make_async_copy(
                    table_hbm.at[pl.ds(0, _SEQ)],
                    r.at[pl.ds(j * _SEQ, _SEQ)],
                    sems[b],
                ).wait()

            def acc_body(t, accs):
                return tuple(
                    accs[i] + r[i * _SEQ + t, :] + r[i * _SEQ + _HALF + t, :]
                    for i in range(_EC)
                )

            accs = lax.fori_loop(
                0, _HALF, acc_body,
                tuple(jnp.zeros((_D,), jnp.float32) for _ in range(_EC)),
            )
            for i in range(_EC):
                pooled_all[ci * _EC + i, :] = accs[i] * (1.0 / _SEQ)

        start_idx(0, 0)
        wait_idx(0)
        start_idx(1, 1)
        for b in range(_NBUF):
            fire(b, b, 0)

        for q in range(4):
            base = q * _QCH
            ib = q % 2

            @pl.loop(0, _QCH - 2 * _NBUF, step=_NBUF)
            def _(cv, base=base, ib=ib):
                for b in range(_NBUF):
                    process(base + cv + b, b)
                    fire(cv + b + _NBUF, b, ib)

            for b in range(_NBUF):
                process(base + _QCH - 2 * _NBUF + b, b)
                fire(_QCH - _NBUF + b, b, ib)

            if q < 3:
                wait_idx(1 - ib)
                for b in range(_NBUF):
                    process(base + _QCH - _NBUF + b, b)
                    fire(b, b, 1 - ib)
                if q < 2:
                    start_idx(q + 2, ib)
            else:
                for b in range(_NBUF):
                    process(base + _QCH - _NBUF + b, b)

        pltpu.sync_copy(pooled_all, out_hbm.at[pl.ds(elem0, elems_per_w)])

    return k(table, x2)


def _mlp(pooled, W1, b1, W2, b2):
    bm = 2048

    def body(p_ref, w1_ref, b1_ref, w2_ref, b2_ref, o_ref):
        h = jnp.dot(p_ref[...], w1_ref[...], preferred_element_type=jnp.float32)
        h = jnp.maximum(h + b1_ref[...], 0.0)
        z = jnp.dot(h, w2_ref[...], preferred_element_type=jnp.float32)
        o_ref[...] = jax.nn.sigmoid(z + b2_ref[...])

    return pl.pallas_call(
        body,
        grid=(_B // bm,),
        in_specs=[
            pl.BlockSpec((bm, _D), lambda i: (i, 0)),
            pl.BlockSpec((_D, _H), lambda i: (0, 0)),
            pl.BlockSpec((1, _H), lambda i: (0, 0)),
            pl.BlockSpec((_H, 1), lambda i: (0, 0)),
            pl.BlockSpec((1, 1), lambda i: (0, 0)),
        ],
        out_specs=pl.BlockSpec((bm, 1), lambda i: (i, 0)),
        out_shape=jax.ShapeDtypeStruct((_B, 1), jnp.float32),
    )(pooled, W1, b1, W2, b2)


@jax.jit
def kernel(x, emb_table, W1, b1, W2, b2):
    pooled = _sc_pool(emb_table, x)
    return _mlp(pooled, W1, b1.reshape(1, _H), W2, b2.reshape(1, 1))
